# speedup vs baseline: 1.9182x; 1.9182x over previous
import jax
import jax.numpy as jnp
from jax.experimental import pallas as pl
from jax.experimental.pallas import tpu as pltpu

_T, _B, _NG, _NP = 100, 256, 2048, 512
_BB = 128


def _dot(a, b):
    return jax.lax.dot_general(
        a, b, (((1,), (0,)), ((), ())), preferred_element_type=jnp.float32
    )


def _rnn_body(v_ref, p0_ref, wenc_ref, wih_ref, whh_ref, wdec_ref, out_ref, h_ref):
    t = pl.program_id(1)

    @pl.when(t == 0)
    def _():
        h_ref[...] = _dot(p0_ref[...], wenc_ref[...])

    pre = _dot(v_ref[0], wih_ref[...]) + _dot(h_ref[...], whh_ref[...])
    h = jnp.tanh(pre)
    h_ref[...] = h
    out_ref[0] = _dot(h, wdec_ref[...])


def kernel(v, p0, W_enc, W_ih, W_hh, W_dec):
    wenc_t = W_enc.T
    wih_t = W_ih.T
    whh_t = W_hh.T
    wdec_t = W_dec.T

    return pl.pallas_call(
        _rnn_body,
        out_shape=jax.ShapeDtypeStruct((_T, _B, _NP), jnp.float32),
        grid=(_B // _BB, _T),
        in_specs=[
            pl.BlockSpec((1, _BB, 2), lambda b, t: (t, b, 0)),
            pl.BlockSpec((_BB, _NP), lambda b, t: (b, 0)),
            pl.BlockSpec((_NP, _NG), lambda b, t: (0, 0)),
            pl.BlockSpec((2, _NG), lambda b, t: (0, 0)),
            pl.BlockSpec((_NG, _NG), lambda b, t: (0, 0)),
            pl.BlockSpec((_NG, _NP), lambda b, t: (0, 0)),
        ],
        out_specs=pl.BlockSpec((1, _BB, _NP), lambda b, t: (t, b, 0)),
        scratch_shapes=[pltpu.VMEM((_BB, _NG), jnp.float32)],
        compiler_params=pltpu.CompilerParams(
            dimension_semantics=("parallel", "arbitrary"),
            vmem_limit_bytes=56 * 1024 * 1024,
        ),
        name="elman_rnn_fused",
    )(v, p0, wenc_t, wih_t, whh_t, wdec_t)


# speedup vs baseline: 2.1423x; 1.1168x over previous
import jax
import jax.numpy as jnp
from jax.experimental import pallas as pl
from jax.experimental.pallas import tpu as pltpu

_T, _B, _NG, _NP = 100, 256, 2048, 512
_BB = 256


def _dot(a, b):
    return jax.lax.dot_general(
        a, b, (((1,), (0,)), ((), ())), preferred_element_type=jnp.float32
    )


def _rnn_body(v_ref, p0_ref, wenc_ref, wih_ref, whh_ref, wdec_ref, out_ref, h_ref):
    t = pl.program_id(1)

    @pl.when(t == 0)
    def _():
        h_ref[...] = _dot(p0_ref[...], wenc_ref[...])

    pre = _dot(v_ref[0], wih_ref[...]) + _dot(h_ref[...], whh_ref[...])
    h = jnp.tanh(pre)
    h_ref[...] = h
    out_ref[0] = _dot(h, wdec_ref[...])


def kernel(v, p0, W_enc, W_ih, W_hh, W_dec):
    wenc_t = W_enc.T
    wih_t = W_ih.T
    whh_t = W_hh.T
    wdec_t = W_dec.T

    return pl.pallas_call(
        _rnn_body,
        out_shape=jax.ShapeDtypeStruct((_T, _B, _NP), jnp.float32),
        grid=(_B // _BB, _T),
        in_specs=[
            pl.BlockSpec((1, _BB, 2), lambda b, t: (t, b, 0)),
            pl.BlockSpec((_BB, _NP), lambda b, t: (b, 0)),
            pl.BlockSpec((_NP, _NG), lambda b, t: (0, 0)),
            pl.BlockSpec((2, _NG), lambda b, t: (0, 0)),
            pl.BlockSpec((_NG, _NG), lambda b, t: (0, 0)),
            pl.BlockSpec((_NG, _NP), lambda b, t: (0, 0)),
        ],
        out_specs=pl.BlockSpec((1, _BB, _NP), lambda b, t: (t, b, 0)),
        scratch_shapes=[pltpu.VMEM((_BB, _NG), jnp.float32)],
        compiler_params=pltpu.CompilerParams(
            dimension_semantics=("parallel", "arbitrary"),
            vmem_limit_bytes=56 * 1024 * 1024,
        ),
        name="elman_rnn_fused",
    )(v, p0, wenc_t, wih_t, whh_t, wdec_t)


# speedup vs baseline: 2.2906x; 1.0692x over previous
import jax
import jax.numpy as jnp
from jax.experimental import pallas as pl
from jax.experimental.pallas import tpu as pltpu

_T, _B, _NG, _NP = 100, 256, 2048, 512
_BB = 256


def _dot(a, b):
    return jax.lax.dot_general(
        a, b, (((1,), (0,)), ((), ())), preferred_element_type=jnp.float32
    )


def _rnn_body(v_ref, p0_ref, wenc_ref, wih_ref, whh_ref, wdec_ref, out_ref, h_ref):
    t = pl.program_id(1)

    @pl.when(t == 0)
    def _():
        h_ref[...] = _dot(p0_ref[...], wenc_ref[...])

    vt = v_ref[0]
    vin = vt[:, 0:1] * wih_ref[0:1, :] + vt[:, 1:2] * wih_ref[1:2, :]
    pre = vin + _dot(h_ref[...], whh_ref[...])
    h = jnp.tanh(pre)
    h_ref[...] = h
    out_ref[0] = _dot(h, wdec_ref[...])


def kernel(v, p0, W_enc, W_ih, W_hh, W_dec):
    wenc_t = W_enc.T
    wih_t = W_ih.T
    whh_t = W_hh.T
    wdec_t = W_dec.T

    return pl.pallas_call(
        _rnn_body,
        out_shape=jax.ShapeDtypeStruct((_T, _B, _NP), jnp.float32),
        grid=(_B // _BB, _T),
        in_specs=[
            pl.BlockSpec((1, _BB, 2), lambda b, t: (t, b, 0)),
            pl.BlockSpec((_BB, _NP), lambda b, t: (b, 0)),
            pl.BlockSpec((_NP, _NG), lambda b, t: (0, 0)),
            pl.BlockSpec((2, _NG), lambda b, t: (0, 0)),
            pl.BlockSpec((_NG, _NG), lambda b, t: (0, 0)),
            pl.BlockSpec((_NG, _NP), lambda b, t: (0, 0)),
        ],
        out_specs=pl.BlockSpec((1, _BB, _NP), lambda b, t: (t, b, 0)),
        scratch_shapes=[pltpu.VMEM((_BB, _NG), jnp.float32)],
        compiler_params=pltpu.CompilerParams(
            dimension_semantics=("parallel", "arbitrary"),
            vmem_limit_bytes=56 * 1024 * 1024,
        ),
        name="elman_rnn_fused",
    )(v, p0, wenc_t, wih_t, whh_t, wdec_t)
